# Initial kernel scaffold; baseline (speedup 1.0000x reference)
import jax
import jax.numpy as jnp
from jax.experimental import pallas as pl


def kernel(x, probe_idx):
    raise NotImplementedError("write your pallas kernel here")


# speedup vs baseline: 2.3791x; 2.3791x over previous
import functools

import jax
import jax.numpy as jnp
from jax.experimental import pallas as pl
from jax.experimental.pallas import tpu as pltpu

_ROWS = 4096
_COLS = 8192
_NPROBE = 128
_BLK = 512


def _gather_kernel(idx_ref, x_ref, o_ref, onehot_ref):
    @pl.when(pl.program_id(0) == 0)
    def _build_onehot():
        idx = idx_ref[0, :]
        cols = jax.lax.broadcasted_iota(jnp.int32, (_COLS, _NPROBE), 0)
        onehot_ref[...] = (cols == idx[None, :]).astype(jnp.bfloat16)

    xb = x_ref[...]
    hi = xb.astype(jnp.bfloat16)
    lo = (xb - hi.astype(jnp.float32)).astype(jnp.bfloat16)
    oh = onehot_ref[...]
    acc = jax.lax.dot_general(
        hi, oh, (((1,), (0,)), ((), ())), preferred_element_type=jnp.float32
    )
    acc += jax.lax.dot_general(
        lo, oh, (((1,), (0,)), ((), ())), preferred_element_type=jnp.float32
    )
    o_ref[...] = acc


@jax.jit
def kernel(x, probe_idx):
    idx2d = probe_idx.reshape(1, _NPROBE).astype(jnp.int32)
    grid = (_ROWS // _BLK,)
    return pl.pallas_call(
        _gather_kernel,
        grid=grid,
        in_specs=[
            pl.BlockSpec((1, _NPROBE), lambda i: (0, 0)),
            pl.BlockSpec((_BLK, _COLS), lambda i: (i, 0)),
        ],
        out_specs=pl.BlockSpec((_BLK, _NPROBE), lambda i: (i, 0)),
        out_shape=jax.ShapeDtypeStruct((_ROWS, _NPROBE), jnp.float32),
        scratch_shapes=[pltpu.VMEM((_COLS, _NPROBE), jnp.bfloat16)],
        compiler_params=pltpu.CompilerParams(
            dimension_semantics=("arbitrary",),
        ),
    )(idx2d, x)
